# Initial kernel scaffold; baseline (speedup 1.0000x reference)
import jax
import jax.numpy as jnp
from jax.experimental import pallas as pl


def kernel(e, s, edge_index, edge_attr, fc_W, attn_W, feat_W, feat_b, w1, b1, w2, b2, ln_g, ln_b):
    raise NotImplementedError("write your pallas kernel here")


# speedup vs baseline: 5.8081x; 5.8081x over previous
import functools
import jax
import jax.numpy as jnp
import numpy as np
from jax import lax
from jax.experimental import pallas as pl
from jax.experimental.pallas import tpu as pltpu
from jax.experimental.pallas import tpu_sc as plsc

N_E = 10000
N_S = 10000
E = 320000
H = 8
DH = 16
FEAT = 50
FEAT_PAD = 64
FFN = 512

NC = 2
NSUB = 16
NW = NC * NSUB
EPW = E // NW
EB = 32
N_SP = 10240
ROWS_PER_TILE = N_SP // NSUB

NODE_BLK = 1000
EDGE_BLK = 3200


def _node_prep_body(e_ref, wcat_ref, a1m_ref, zc_ref, ns_ref):
    z = jnp.dot(e_ref[...], wcat_ref[...], preferred_element_type=jnp.float32)
    zc_ref[...] = z
    ns_ref[...] = jnp.dot(z, a1m_ref[...], preferred_element_type=jnp.float32)


def _node_prep(e, wcat, a1m):
    return pl.pallas_call(
        _node_prep_body,
        grid=(N_E // NODE_BLK,),
        in_specs=[
            pl.BlockSpec((NODE_BLK, 128), lambda i: (i, 0)),
            pl.BlockSpec((128, 128), lambda i: (0, 0)),
            pl.BlockSpec((128, 16), lambda i: (0, 0)),
        ],
        out_specs=[
            pl.BlockSpec((NODE_BLK, 128), lambda i: (i, 0)),
            pl.BlockSpec((NODE_BLK, 16), lambda i: (i, 0)),
        ],
        out_shape=[
            jax.ShapeDtypeStruct((N_E, 128), jnp.float32),
            jax.ShapeDtypeStruct((N_E, 16), jnp.float32),
        ],
    )(e, wcat, a1m)


def _edge_prep_body(ea_ref, c_ref, d_ref, ps_ref):
    ps_ref[...] = (
        jnp.dot(ea_ref[...], c_ref[...], preferred_element_type=jnp.float32)
        + d_ref[...]
    )


def _edge_prep(ea_pad, cpad, dvec):
    return pl.pallas_call(
        _edge_prep_body,
        grid=(E // EDGE_BLK,),
        in_specs=[
            pl.BlockSpec((EDGE_BLK, FEAT_PAD), lambda i: (i, 0)),
            pl.BlockSpec((FEAT_PAD, 16), lambda i: (0, 0)),
            pl.BlockSpec((1, 16), lambda i: (0, 0)),
        ],
        out_specs=pl.BlockSpec((EDGE_BLK, 16), lambda i: (i, 0)),
        out_shape=jax.ShapeDtypeStruct((E, 16), jnp.float32),
    )(ea_pad, cpad, dvec)


NBATCH = E // EB
NB_BASE = NBATCH // NW
NB_EXTRA = NBATCH - NB_BASE * NW


def _sc_edge_body(zc_hbm, nsp_hbm, ps1_hbm, src_hbm, dst_hbm, zagg_hbm,
                  aggp_hbm, denp_hbm,
                  srcb, srcb8, dstb, dstb8, zrows, nsg, psrows, wbuf,
                  agg_sh, den_sh, nst_sh, sem1, sem2):
    cid = lax.axis_index("c")
    sid = lax.axis_index("s")
    wid = cid * NSUB + sid
    zbase = sid * ROWS_PER_TILE
    pbase = sid * (ROWS_PER_TILE // 8)

    pltpu.sync_copy(zagg_hbm.at[pl.ds(zbase, ROWS_PER_TILE)],
                    agg_sh.at[pl.ds(zbase, ROWS_PER_TILE)])
    pltpu.sync_copy(zagg_hbm.at[pl.ds(zbase, ROWS_PER_TILE // 8)],
                    den_sh.at[pl.ds(pbase, ROWS_PER_TILE // 8)])
    pltpu.sync_copy(nsp_hbm.at[pl.ds(pbase, ROWS_PER_TILE // 8)],
                    nst_sh.at[pl.ds(pbase, ROWS_PER_TILE // 8)])

    def zw(i, c):
        for slot in range(8):
            wbuf[i, pl.ds(slot * 16, 16)] = jnp.zeros((16,), jnp.float32)
        return c
    lax.fori_loop(0, EB, zw, 0)
    plsc.subcore_barrier()

    nb = jnp.where(wid < NB_EXTRA, NB_BASE + 1, NB_BASE)

    def batch_body(t, carry):
        off = (t * NW + wid) * EB
        pltpu.sync_copy(src_hbm.at[pl.ds(off, EB)], srcb)
        pltpu.sync_copy(dst_hbm.at[pl.ds(off, EB)], dstb)
        pltpu.sync_copy(ps1_hbm.at[pl.ds(off * 16, EB * 16)], psrows)
        for k in range(EB // 16):
            srcb8[pl.ds(k * 16, 16)] = lax.shift_right_logical(
                srcb[pl.ds(k * 16, 16)], 3)
            dstb8[pl.ds(k * 16, 16)] = lax.shift_right_logical(
                dstb[pl.ds(k * 16, 16)], 3)
        cp1 = pltpu.async_copy(zc_hbm.at[srcb], zrows, sem1)
        cp2 = pltpu.async_copy(nst_sh.at[srcb8], nsg, sem2)
        cp1.wait()
        cp2.wait()

        def group_body(g, c1):
            sv = srcb[pl.ds(g * 16, 16)]
            dv = dstb[pl.ds(g * 16, 16)]
            for l in range(16):
                i = g * 16 + l
                soff = (sv[l] & 7) * 16
                doff = (dv[l] & 7) * 16
                nsv = nsg[i, pl.ds(soff, 16)]
                scv = nsv + psrows[pl.ds(i * 16, 16)]
                scv = jnp.where(scv >= 0.0, scv, 0.01 * scv)
                wv = jnp.exp(scv)
                wbuf[i, pl.ds(doff, 16)] = wv
                for h in range(H):
                    seg = zrows[i, pl.ds(h * 16, 16)]
                    zrows[i, pl.ds(h * 16, 16)] = seg * wv[h]
            return c1

        lax.fori_loop(0, EB // 16, group_body, 0)
        pltpu.sync_copy(zrows, agg_sh.at[dstb], add=True)
        pltpu.sync_copy(wbuf, den_sh.at[dstb8], add=True)

        def rz(g, c2):
            dv = dstb[pl.ds(g * 16, 16)]
            for l in range(16):
                wbuf[g * 16 + l, pl.ds((dv[l] & 7) * 16, 16)] = (
                    jnp.zeros((16,), jnp.float32))
            return c2

        lax.fori_loop(0, EB // 16, rz, 0)
        return carry

    lax.fori_loop(0, nb, batch_body, 0)
    plsc.subcore_barrier()

    pltpu.sync_copy(agg_sh.at[pl.ds(zbase, ROWS_PER_TILE)],
                    aggp_hbm.at[cid, pl.ds(zbase, ROWS_PER_TILE)])
    pltpu.sync_copy(den_sh.at[pl.ds(pbase, ROWS_PER_TILE // 8)],
                    denp_hbm.at[cid, pl.ds(pbase, ROWS_PER_TILE // 8)])


def _sc_edge(zc, nsp128, ps1, src, dst, zagg):
    mesh = plsc.VectorSubcoreMesh(core_axis_name="c", subcore_axis_name="s")
    fn = functools.partial(
        pl.kernel,
        mesh=mesh,
        out_type=[
            jax.ShapeDtypeStruct((NC, N_SP, 128), jnp.float32),
            jax.ShapeDtypeStruct((NC, N_SP // 8, 128), jnp.float32),
        ],
        scratch_types=[
            pltpu.VMEM((EB,), jnp.int32),
            pltpu.VMEM((EB,), jnp.int32),
            pltpu.VMEM((EB,), jnp.int32),
            pltpu.VMEM((EB,), jnp.int32),
            pltpu.VMEM((EB, 128), jnp.float32),
            pltpu.VMEM((EB, 128), jnp.float32),
            pltpu.VMEM((EB * 16,), jnp.float32),
            pltpu.VMEM((EB, 128), jnp.float32),
            pltpu.VMEM_SHARED((N_SP, 128), jnp.float32),
            pltpu.VMEM_SHARED((N_SP // 8, 128), jnp.float32),
            pltpu.VMEM_SHARED((N_SP // 8, 128), jnp.float32),
            pltpu.SemaphoreType.DMA,
            pltpu.SemaphoreType.DMA,
        ],
    )(_sc_edge_body)
    return fn(zc, nsp128, ps1, src, dst, zagg)


def _tail_body(a0_ref, a1_ref, d0_ref, d1_ref, s_ref, exp_ref,
               w1_ref, b1_ref, w2_ref, b2_ref, g_ref, b_ref, out_ref):
    aggu = a0_ref[...] + a1_ref[...]
    den = d0_ref[...] + d1_ref[...] + 1e-10
    den_exp = jnp.dot(den, exp_ref[...], preferred_element_type=jnp.float32)
    agg = aggu / den_exp
    hh = jnp.where(agg > 0.0, agg, jnp.exp(agg) - 1.0) + s_ref[...]
    mu = jnp.mean(hh, axis=-1, keepdims=True)
    var = jnp.mean((hh - mu) ** 2, axis=-1, keepdims=True)
    xn = (hh - mu) * lax.rsqrt(var + 1e-6) * g_ref[...] + b_ref[...]
    t = jnp.dot(xn, w1_ref[...], preferred_element_type=jnp.float32) + b1_ref[...]
    inter = t * 0.5 * (1.0 + lax.erf(t * np.float32(1.0 / np.sqrt(2.0))))
    out_ref[...] = (
        jnp.dot(inter, w2_ref[...], preferred_element_type=jnp.float32)
        + b2_ref[...] + hh
    )


def _tail(a0, a1, d0, d1, s, expand, w1, b1, w2, b2, ln_g, ln_b):
    return pl.pallas_call(
        _tail_body,
        grid=(N_S // NODE_BLK,),
        in_specs=[
            pl.BlockSpec((NODE_BLK, 128), lambda i: (i, 0)),
            pl.BlockSpec((NODE_BLK, 128), lambda i: (i, 0)),
            pl.BlockSpec((NODE_BLK, 16), lambda i: (i, 0)),
            pl.BlockSpec((NODE_BLK, 16), lambda i: (i, 0)),
            pl.BlockSpec((NODE_BLK, 128), lambda i: (i, 0)),
            pl.BlockSpec((16, 128), lambda i: (0, 0)),
            pl.BlockSpec((128, FFN), lambda i: (0, 0)),
            pl.BlockSpec((1, FFN), lambda i: (0, 0)),
            pl.BlockSpec((FFN, 128), lambda i: (0, 0)),
            pl.BlockSpec((1, 128), lambda i: (0, 0)),
            pl.BlockSpec((1, 128), lambda i: (0, 0)),
            pl.BlockSpec((1, 128), lambda i: (0, 0)),
        ],
        out_specs=pl.BlockSpec((NODE_BLK, 128), lambda i: (i, 0)),
        out_shape=jax.ShapeDtypeStruct((N_S, 128), jnp.float32),
    )(a0, a1, d0, d1, s, expand, w1, b1, w2, b2, ln_g, ln_b)


def kernel(e, s, edge_index, edge_attr, fc_W, attn_W, feat_W, feat_b,
           w1, b1, w2, b2, ln_g, ln_b):
    src = edge_index[0].astype(jnp.int32)
    dst = edge_index[1].astype(jnp.int32)
    a1 = attn_W[:, :DH]
    a2 = attn_W[:, DH:]

    wcat = jnp.transpose(fc_W, (1, 0, 2)).reshape(128, H * DH)
    a1m = jnp.zeros((128, 16), jnp.float32).at[
        jnp.arange(128), jnp.arange(128) // 16].set(a1.reshape(-1))
    c = jnp.einsum('hfk,hk->fh', feat_W, a2)
    cpad = jnp.zeros((FEAT_PAD, 16), jnp.float32).at[:FEAT, :H].set(c)
    dvec = jnp.zeros((1, 16), jnp.float32).at[0, :H].set(
        jnp.einsum('hk,hk->h', feat_b, a2))
    ea_pad = jnp.pad(edge_attr, ((0, 0), (0, FEAT_PAD - FEAT)))
    expand = jnp.zeros((16, 128), jnp.float32).at[
        jnp.arange(128) // 16, jnp.arange(128)].set(1.0)

    zc, ns = _node_prep(e, wcat, a1m)
    ps = _edge_prep(ea_pad, cpad, dvec)

    zagg = jnp.zeros((N_SP, 128), jnp.float32)
    nsp128 = jnp.pad(ns, ((0, N_SP - N_E), (0, 0))).reshape(N_SP // 8, 128)
    ps1 = ps.reshape(-1)
    aggp, denp128 = _sc_edge(zc, nsp128, ps1, src, dst, zagg)
    denp = denp128.reshape(NC, N_SP, 16)

    return _tail(aggp[0, :N_S], aggp[1, :N_S], denp[0, :N_S], denp[1, :N_S],
                 s, expand,
                 w1, b1.reshape(1, FFN), w2, b2.reshape(1, 128),
                 ln_g.reshape(1, 128), ln_b.reshape(1, 128))


# speedup vs baseline: 6.9396x; 1.1948x over previous
import functools
import jax
import jax.numpy as jnp
import numpy as np
from jax import lax
from jax.experimental import pallas as pl
from jax.experimental.pallas import tpu as pltpu
from jax.experimental.pallas import tpu_sc as plsc

N_E = 10000
N_S = 10000
E = 320000
H = 8
DH = 16
FEAT = 50
FEAT_PAD = 64
FFN = 512

NC = 2
NSUB = 16
NW = NC * NSUB
EPW = E // NW
EB = 32
N_SP = 10240
ROWS_PER_TILE = N_SP // NSUB

NODE_BLK = 1000
EDGE_BLK = 3200


def _node_prep_body(e_ref, wcat_ref, a1m_ref, zc_ref, ns_ref):
    z = jnp.dot(e_ref[...], wcat_ref[...], preferred_element_type=jnp.float32)
    zc_ref[...] = z
    ns_ref[...] = jnp.dot(z, a1m_ref[...], preferred_element_type=jnp.float32)


def _node_prep(e, wcat, a1m):
    return pl.pallas_call(
        _node_prep_body,
        grid=(N_E // NODE_BLK,),
        in_specs=[
            pl.BlockSpec((NODE_BLK, 128), lambda i: (i, 0)),
            pl.BlockSpec((128, 128), lambda i: (0, 0)),
            pl.BlockSpec((128, 16), lambda i: (0, 0)),
        ],
        out_specs=[
            pl.BlockSpec((NODE_BLK, 128), lambda i: (i, 0)),
            pl.BlockSpec((NODE_BLK, 16), lambda i: (i, 0)),
        ],
        out_shape=[
            jax.ShapeDtypeStruct((N_E, 128), jnp.float32),
            jax.ShapeDtypeStruct((N_E, 16), jnp.float32),
        ],
    )(e, wcat, a1m)


def _edge_prep_body(ea_ref, c_ref, d_ref, ps_ref):
    ps_ref[...] = (
        jnp.dot(ea_ref[...], c_ref[...], preferred_element_type=jnp.float32)
        + d_ref[...]
    )


def _edge_prep(ea_pad, cpad, dvec):
    return pl.pallas_call(
        _edge_prep_body,
        grid=(E // EDGE_BLK,),
        in_specs=[
            pl.BlockSpec((EDGE_BLK, FEAT_PAD), lambda i: (i, 0)),
            pl.BlockSpec((FEAT_PAD, 16), lambda i: (0, 0)),
            pl.BlockSpec((1, 16), lambda i: (0, 0)),
        ],
        out_specs=pl.BlockSpec((EDGE_BLK, 16), lambda i: (i, 0)),
        out_shape=jax.ShapeDtypeStruct((E, 16), jnp.float32),
    )(ea_pad, cpad, dvec)


NBATCH = E // EB
NB_BASE = NBATCH // NW
NB_EXTRA = NBATCH - NB_BASE * NW


def _sc_edge_body(zc_hbm, nsp_hbm, ps1_hbm, src_hbm, dst_hbm, zagg_hbm,
                  aggp_hbm, denp_hbm,
                  srcb, srcb8, dstb, dstb8, zrows, nsg, psrows, wbuf,
                  agg_sh, den_sh, nst_sh, sem1, sem2):
    cid = lax.axis_index("c")
    sid = lax.axis_index("s")
    wid = cid * NSUB + sid
    zbase = sid * ROWS_PER_TILE
    pbase = sid * (ROWS_PER_TILE // 8)

    pltpu.sync_copy(zagg_hbm.at[pl.ds(zbase, ROWS_PER_TILE)],
                    agg_sh.at[pl.ds(zbase, ROWS_PER_TILE)])
    pltpu.sync_copy(zagg_hbm.at[pl.ds(zbase, ROWS_PER_TILE // 8)],
                    den_sh.at[pl.ds(pbase, ROWS_PER_TILE // 8)])
    pltpu.sync_copy(nsp_hbm.at[pl.ds(pbase, ROWS_PER_TILE // 8)],
                    nst_sh.at[pl.ds(pbase, ROWS_PER_TILE // 8)])

    def zw(i, c):
        for slot in range(8):
            wbuf[i, pl.ds(slot * 16, 16)] = jnp.zeros((16,), jnp.float32)
        return c
    lax.fori_loop(0, EB, zw, 0)
    plsc.subcore_barrier()

    nb = jnp.where(wid < NB_EXTRA, NB_BASE + 1, NB_BASE)

    def batch_body(t, carry):
        off = (t * NW + wid) * EB
        ca1 = pltpu.async_copy(src_hbm.at[pl.ds(off, EB)], srcb, sem1)
        ca2 = pltpu.async_copy(dst_hbm.at[pl.ds(off, EB)], dstb, sem2)
        ca3 = pltpu.async_copy(ps1_hbm.at[pl.ds(off * 16, EB * 16)], psrows, sem1)
        ca1.wait()
        ca2.wait()
        ca3.wait()
        for k in range(EB // 16):
            srcb8[pl.ds(k * 16, 16)] = lax.shift_right_logical(
                srcb[pl.ds(k * 16, 16)], 3)
            dstb8[pl.ds(k * 16, 16)] = lax.shift_right_logical(
                dstb[pl.ds(k * 16, 16)], 3)
        cp1 = pltpu.async_copy(zc_hbm.at[srcb], zrows, sem1)
        cp2 = pltpu.async_copy(nst_sh.at[srcb8], nsg, sem2)
        cp1.wait()
        cp2.wait()

        def group_body(g, c1):
            sv = srcb[pl.ds(g * 16, 16)]
            dv = dstb[pl.ds(g * 16, 16)]
            for l in range(16):
                i = g * 16 + l
                soff = (sv[l] & 7) * 16
                doff = (dv[l] & 7) * 16
                nsv = nsg[i, pl.ds(soff, 16)]
                scv = nsv + psrows[pl.ds(i * 16, 16)]
                scv = jnp.where(scv >= 0.0, scv, 0.01 * scv)
                wv = jnp.exp(scv)
                wbuf[i, pl.ds(doff, 16)] = wv
                for h in range(H):
                    seg = zrows[i, pl.ds(h * 16, 16)]
                    zrows[i, pl.ds(h * 16, 16)] = seg * wv[h]
            return c1

        lax.fori_loop(0, EB // 16, group_body, 0)
        pltpu.sync_copy(zrows, agg_sh.at[dstb], add=True)
        pltpu.sync_copy(wbuf, den_sh.at[dstb8], add=True)

        def rz(g, c2):
            dv = dstb[pl.ds(g * 16, 16)]
            for l in range(16):
                wbuf[g * 16 + l, pl.ds((dv[l] & 7) * 16, 16)] = (
                    jnp.zeros((16,), jnp.float32))
            return c2

        lax.fori_loop(0, EB // 16, rz, 0)
        return carry

    lax.fori_loop(0, nb, batch_body, 0)
    plsc.subcore_barrier()

    pltpu.sync_copy(agg_sh.at[pl.ds(zbase, ROWS_PER_TILE)],
                    aggp_hbm.at[cid, pl.ds(zbase, ROWS_PER_TILE)])
    pltpu.sync_copy(den_sh.at[pl.ds(pbase, ROWS_PER_TILE // 8)],
                    denp_hbm.at[cid, pl.ds(pbase, ROWS_PER_TILE // 8)])


def _sc_edge(zc, nsp128, ps1, src, dst, zagg):
    mesh = plsc.VectorSubcoreMesh(core_axis_name="c", subcore_axis_name="s")
    fn = functools.partial(
        pl.kernel,
        mesh=mesh,
        out_type=[
            jax.ShapeDtypeStruct((NC, N_SP, 128), jnp.float32),
            jax.ShapeDtypeStruct((NC, N_SP // 8, 128), jnp.float32),
        ],
        scratch_types=[
            pltpu.VMEM((EB,), jnp.int32),
            pltpu.VMEM((EB,), jnp.int32),
            pltpu.VMEM((EB,), jnp.int32),
            pltpu.VMEM((EB,), jnp.int32),
            pltpu.VMEM((EB, 128), jnp.float32),
            pltpu.VMEM((EB, 128), jnp.float32),
            pltpu.VMEM((EB * 16,), jnp.float32),
            pltpu.VMEM((EB, 128), jnp.float32),
            pltpu.VMEM_SHARED((N_SP, 128), jnp.float32),
            pltpu.VMEM_SHARED((N_SP // 8, 128), jnp.float32),
            pltpu.VMEM_SHARED((N_SP // 8, 128), jnp.float32),
            pltpu.SemaphoreType.DMA,
            pltpu.SemaphoreType.DMA,
        ],
    )(_sc_edge_body)
    return fn(zc, nsp128, ps1, src, dst, zagg)


def _tail_body(a0_ref, a1_ref, d0_ref, d1_ref, s_ref, exp_ref,
               w1_ref, b1_ref, w2_ref, b2_ref, g_ref, b_ref, out_ref):
    aggu = a0_ref[...] + a1_ref[...]
    den = d0_ref[...] + d1_ref[...] + 1e-10
    den_exp = jnp.dot(den, exp_ref[...], preferred_element_type=jnp.float32)
    agg = aggu / den_exp
    hh = jnp.where(agg > 0.0, agg, jnp.exp(agg) - 1.0) + s_ref[...]
    mu = jnp.mean(hh, axis=-1, keepdims=True)
    var = jnp.mean((hh - mu) ** 2, axis=-1, keepdims=True)
    xn = (hh - mu) * lax.rsqrt(var + 1e-6) * g_ref[...] + b_ref[...]
    t = jnp.dot(xn, w1_ref[...], preferred_element_type=jnp.float32) + b1_ref[...]
    inter = t * 0.5 * (1.0 + lax.erf(t * np.float32(1.0 / np.sqrt(2.0))))
    out_ref[...] = (
        jnp.dot(inter, w2_ref[...], preferred_element_type=jnp.float32)
        + b2_ref[...] + hh
    )


def _tail(a0, a1, d0, d1, s, expand, w1, b1, w2, b2, ln_g, ln_b):
    return pl.pallas_call(
        _tail_body,
        grid=(N_S // NODE_BLK,),
        in_specs=[
            pl.BlockSpec((NODE_BLK, 128), lambda i: (i, 0)),
            pl.BlockSpec((NODE_BLK, 128), lambda i: (i, 0)),
            pl.BlockSpec((NODE_BLK, 16), lambda i: (i, 0)),
            pl.BlockSpec((NODE_BLK, 16), lambda i: (i, 0)),
            pl.BlockSpec((NODE_BLK, 128), lambda i: (i, 0)),
            pl.BlockSpec((16, 128), lambda i: (0, 0)),
            pl.BlockSpec((128, FFN), lambda i: (0, 0)),
            pl.BlockSpec((1, FFN), lambda i: (0, 0)),
            pl.BlockSpec((FFN, 128), lambda i: (0, 0)),
            pl.BlockSpec((1, 128), lambda i: (0, 0)),
            pl.BlockSpec((1, 128), lambda i: (0, 0)),
            pl.BlockSpec((1, 128), lambda i: (0, 0)),
        ],
        out_specs=pl.BlockSpec((NODE_BLK, 128), lambda i: (i, 0)),
        out_shape=jax.ShapeDtypeStruct((N_S, 128), jnp.float32),
    )(a0, a1, d0, d1, s, expand, w1, b1, w2, b2, ln_g, ln_b)


def kernel(e, s, edge_index, edge_attr, fc_W, attn_W, feat_W, feat_b,
           w1, b1, w2, b2, ln_g, ln_b):
    src = edge_index[0].astype(jnp.int32)
    dst = edge_index[1].astype(jnp.int32)
    a1 = attn_W[:, :DH]
    a2 = attn_W[:, DH:]

    wcat = jnp.transpose(fc_W, (1, 0, 2)).reshape(128, H * DH)
    a1m = jnp.zeros((128, 16), jnp.float32).at[
        jnp.arange(128), jnp.arange(128) // 16].set(a1.reshape(-1))
    c = jnp.einsum('hfk,hk->fh', feat_W, a2)
    cpad = jnp.zeros((FEAT_PAD, 16), jnp.float32).at[:FEAT, :H].set(c)
    dvec = jnp.zeros((1, 16), jnp.float32).at[0, :H].set(
        jnp.einsum('hk,hk->h', feat_b, a2))
    ea_pad = jnp.pad(edge_attr, ((0, 0), (0, FEAT_PAD - FEAT)))
    expand = jnp.zeros((16, 128), jnp.float32).at[
        jnp.arange(128) // 16, jnp.arange(128)].set(1.0)

    zc, ns = _node_prep(e, wcat, a1m)
    ps = _edge_prep(ea_pad, cpad, dvec)

    zagg = jnp.zeros((N_SP, 128), jnp.float32)
    nsp128 = jnp.pad(ns, ((0, N_SP - N_E), (0, 0))).reshape(N_SP // 8, 128)
    ps1 = ps.reshape(-1)
    aggp, denp128 = _sc_edge(zc, nsp128, ps1, src, dst, zagg)
    denp = denp128.reshape(NC, N_SP, 16)

    return _tail(aggp[0, :N_S], aggp[1, :N_S], denp[0, :N_S], denp[1, :N_S],
                 s, expand,
                 w1, b1.reshape(1, FFN), w2, b2.reshape(1, 128),
                 ln_g.reshape(1, 128), ln_b.reshape(1, 128))


# speedup vs baseline: 7.0542x; 1.0165x over previous
import functools
import jax
import jax.numpy as jnp
import numpy as np
from jax import lax
from jax.experimental import pallas as pl
from jax.experimental.pallas import tpu as pltpu
from jax.experimental.pallas import tpu_sc as plsc

N_E = 10000
N_S = 10000
E = 320000
H = 8
DH = 16
FEAT = 50
FEAT_PAD = 64
FFN = 512

NC = 2
NSUB = 16
NW = NC * NSUB
EPW = E // NW
EB = 32
N_SP = 10240
ROWS_PER_TILE = N_SP // NSUB

NODE_BLK = 1000
EDGE_BLK = 3200


def _node_prep_body(e_ref, wcat_ref, a1m_ref, zc_ref, ns_ref):
    z = jnp.dot(e_ref[...], wcat_ref[...], preferred_element_type=jnp.float32)
    zc_ref[...] = z
    ns_ref[...] = jnp.dot(z, a1m_ref[...], preferred_element_type=jnp.float32)


def _node_prep(e, wcat, a1m):
    return pl.pallas_call(
        _node_prep_body,
        grid=(N_E // NODE_BLK,),
        in_specs=[
            pl.BlockSpec((NODE_BLK, 128), lambda i: (i, 0)),
            pl.BlockSpec((128, 128), lambda i: (0, 0)),
            pl.BlockSpec((128, 16), lambda i: (0, 0)),
        ],
        out_specs=[
            pl.BlockSpec((NODE_BLK, 128), lambda i: (i, 0)),
            pl.BlockSpec((NODE_BLK, 16), lambda i: (i, 0)),
        ],
        out_shape=[
            jax.ShapeDtypeStruct((N_E, 128), jnp.float32),
            jax.ShapeDtypeStruct((N_E, 16), jnp.float32),
        ],
    )(e, wcat, a1m)


def _edge_prep_body(ea_ref, c_ref, d_ref, ps_ref):
    ps_ref[...] = (
        jnp.dot(ea_ref[...], c_ref[...], preferred_element_type=jnp.float32)
        + d_ref[...]
    )


def _edge_prep(ea_pad, cpad, dvec):
    return pl.pallas_call(
        _edge_prep_body,
        grid=(E // EDGE_BLK,),
        in_specs=[
            pl.BlockSpec((EDGE_BLK, FEAT_PAD), lambda i: (i, 0)),
            pl.BlockSpec((FEAT_PAD, 16), lambda i: (0, 0)),
            pl.BlockSpec((1, 16), lambda i: (0, 0)),
        ],
        out_specs=pl.BlockSpec((EDGE_BLK, 16), lambda i: (i, 0)),
        out_shape=jax.ShapeDtypeStruct((E, 16), jnp.float32),
    )(ea_pad, cpad, dvec)


NBATCH = E // EB
NB_BASE = NBATCH // NW
NB_EXTRA = NBATCH - NB_BASE * NW


def _sc_edge_body(zc_hbm, nsp_hbm, ps1_hbm, src_hbm, dst_hbm, zagg_hbm,
                  aggp_hbm, denp_hbm,
                  srcb, srcb8, dstb, dstb8, zrows, nsg, psrows, wbuf,
                  agg_sh, den_sh, nst_sh, sem1, sem2):
    cid = lax.axis_index("c")
    sid = lax.axis_index("s")
    wid = cid * NSUB + sid
    zbase = sid * ROWS_PER_TILE
    pbase = sid * (ROWS_PER_TILE // 8)

    pltpu.sync_copy(zagg_hbm.at[pl.ds(zbase, ROWS_PER_TILE)],
                    agg_sh.at[pl.ds(zbase, ROWS_PER_TILE)])
    pltpu.sync_copy(zagg_hbm.at[pl.ds(zbase, ROWS_PER_TILE // 8)],
                    den_sh.at[pl.ds(pbase, ROWS_PER_TILE // 8)])
    pltpu.sync_copy(nsp_hbm.at[pl.ds(pbase, ROWS_PER_TILE // 8)],
                    nst_sh.at[pl.ds(pbase, ROWS_PER_TILE // 8)])

    def zw(i, c):
        for slot in range(8):
            wbuf[i, pl.ds(slot * 16, 16)] = jnp.zeros((16,), jnp.float32)
        return c
    lax.fori_loop(0, EB, zw, 0)
    plsc.subcore_barrier()

    nb = jnp.where(wid < NB_EXTRA, NB_BASE + 1, NB_BASE)

    def batch_body(t, carry):
        off = (t * NW + wid) * EB
        ca1 = pltpu.async_copy(src_hbm.at[pl.ds(off, EB)], srcb, sem1)
        ca2 = pltpu.async_copy(dst_hbm.at[pl.ds(off, EB)], dstb, sem2)
        ca3 = pltpu.async_copy(ps1_hbm.at[pl.ds(off * 16, EB * 16)], psrows, sem1)
        ca1.wait()
        ca2.wait()
        ca3.wait()
        for k in range(EB // 16):
            srcb8[pl.ds(k * 16, 16)] = lax.shift_right_logical(
                srcb[pl.ds(k * 16, 16)], 3)
            dstb8[pl.ds(k * 16, 16)] = lax.shift_right_logical(
                dstb[pl.ds(k * 16, 16)], 3)
        cp1 = pltpu.async_copy(zc_hbm.at[srcb], zrows, sem1)
        cp2 = pltpu.async_copy(nst_sh.at[srcb8], nsg, sem2)
        cp1.wait()
        cp2.wait()

        def group_body(g, c1):
            sv = srcb[pl.ds(g * 16, 16)]
            dv = dstb[pl.ds(g * 16, 16)]
            for l in range(16):
                i = g * 16 + l
                soff = (sv[l] & 7) * 16
                doff = (dv[l] & 7) * 16
                nsv = nsg[i, pl.ds(soff, 16)]
                scv = nsv + psrows[pl.ds(i * 16, 16)]
                scv = jnp.where(scv >= 0.0, scv, 0.01 * scv)
                wv = jnp.exp(scv)
                wbuf[i, pl.ds(doff, 16)] = wv
                for h in range(H):
                    seg = zrows[i, pl.ds(h * 16, 16)]
                    zrows[i, pl.ds(h * 16, 16)] = seg * wv[h]
            return c1

        lax.fori_loop(0, EB // 16, group_body, 0)
        cs1 = pltpu.async_copy(zrows, agg_sh.at[dstb], sem1, add=True)
        cs2 = pltpu.async_copy(wbuf, den_sh.at[dstb8], sem2, add=True)
        cs1.wait()
        cs2.wait()

        def rz(g, c2):
            dv = dstb[pl.ds(g * 16, 16)]
            for l in range(16):
                wbuf[g * 16 + l, pl.ds((dv[l] & 7) * 16, 16)] = (
                    jnp.zeros((16,), jnp.float32))
            return c2

        lax.fori_loop(0, EB // 16, rz, 0)
        return carry

    lax.fori_loop(0, nb, batch_body, 0)
    plsc.subcore_barrier()

    pltpu.sync_copy(agg_sh.at[pl.ds(zbase, ROWS_PER_TILE)],
                    aggp_hbm.at[cid, pl.ds(zbase, ROWS_PER_TILE)])
    pltpu.sync_copy(den_sh.at[pl.ds(pbase, ROWS_PER_TILE // 8)],
                    denp_hbm.at[cid, pl.ds(pbase, ROWS_PER_TILE // 8)])


def _sc_edge(zc, nsp128, ps1, src, dst, zagg):
    mesh = plsc.VectorSubcoreMesh(core_axis_name="c", subcore_axis_name="s")
    fn = functools.partial(
        pl.kernel,
        mesh=mesh,
        out_type=[
            jax.ShapeDtypeStruct((NC, N_SP, 128), jnp.float32),
            jax.ShapeDtypeStruct((NC, N_SP // 8, 128), jnp.float32),
        ],
        scratch_types=[
            pltpu.VMEM((EB,), jnp.int32),
            pltpu.VMEM((EB,), jnp.int32),
            pltpu.VMEM((EB,), jnp.int32),
            pltpu.VMEM((EB,), jnp.int32),
            pltpu.VMEM((EB, 128), jnp.float32),
            pltpu.VMEM((EB, 128), jnp.float32),
            pltpu.VMEM((EB * 16,), jnp.float32),
            pltpu.VMEM((EB, 128), jnp.float32),
            pltpu.VMEM_SHARED((N_SP, 128), jnp.float32),
            pltpu.VMEM_SHARED((N_SP // 8, 128), jnp.float32),
            pltpu.VMEM_SHARED((N_SP // 8, 128), jnp.float32),
            pltpu.SemaphoreType.DMA,
            pltpu.SemaphoreType.DMA,
        ],
    )(_sc_edge_body)
    return fn(zc, nsp128, ps1, src, dst, zagg)


def _tail_body(a0_ref, a1_ref, d0_ref, d1_ref, s_ref, exp_ref,
               w1_ref, b1_ref, w2_ref, b2_ref, g_ref, b_ref, out_ref):
    aggu = a0_ref[...] + a1_ref[...]
    den = d0_ref[...] + d1_ref[...] + 1e-10
    den_exp = jnp.dot(den, exp_ref[...], preferred_element_type=jnp.float32)
    agg = aggu / den_exp
    hh = jnp.where(agg > 0.0, agg, jnp.exp(agg) - 1.0) + s_ref[...]
    mu = jnp.mean(hh, axis=-1, keepdims=True)
    var = jnp.mean((hh - mu) ** 2, axis=-1, keepdims=True)
    xn = (hh - mu) * lax.rsqrt(var + 1e-6) * g_ref[...] + b_ref[...]
    t = jnp.dot(xn, w1_ref[...], preferred_element_type=jnp.float32) + b1_ref[...]
    inter = t * 0.5 * (1.0 + lax.erf(t * np.float32(1.0 / np.sqrt(2.0))))
    out_ref[...] = (
        jnp.dot(inter, w2_ref[...], preferred_element_type=jnp.float32)
        + b2_ref[...] + hh
    )


def _tail(a0, a1, d0, d1, s, expand, w1, b1, w2, b2, ln_g, ln_b):
    return pl.pallas_call(
        _tail_body,
        grid=(N_S // NODE_BLK,),
        in_specs=[
            pl.BlockSpec((NODE_BLK, 128), lambda i: (i, 0)),
            pl.BlockSpec((NODE_BLK, 128), lambda i: (i, 0)),
            pl.BlockSpec((NODE_BLK, 16), lambda i: (i, 0)),
            pl.BlockSpec((NODE_BLK, 16), lambda i: (i, 0)),
            pl.BlockSpec((NODE_BLK, 128), lambda i: (i, 0)),
            pl.BlockSpec((16, 128), lambda i: (0, 0)),
            pl.BlockSpec((128, FFN), lambda i: (0, 0)),
            pl.BlockSpec((1, FFN), lambda i: (0, 0)),
            pl.BlockSpec((FFN, 128), lambda i: (0, 0)),
            pl.BlockSpec((1, 128), lambda i: (0, 0)),
            pl.BlockSpec((1, 128), lambda i: (0, 0)),
            pl.BlockSpec((1, 128), lambda i: (0, 0)),
        ],
        out_specs=pl.BlockSpec((NODE_BLK, 128), lambda i: (i, 0)),
        out_shape=jax.ShapeDtypeStruct((N_S, 128), jnp.float32),
    )(a0, a1, d0, d1, s, expand, w1, b1, w2, b2, ln_g, ln_b)


def kernel(e, s, edge_index, edge_attr, fc_W, attn_W, feat_W, feat_b,
           w1, b1, w2, b2, ln_g, ln_b):
    src = edge_index[0].astype(jnp.int32)
    dst = edge_index[1].astype(jnp.int32)
    a1 = attn_W[:, :DH]
    a2 = attn_W[:, DH:]

    wcat = jnp.transpose(fc_W, (1, 0, 2)).reshape(128, H * DH)
    a1m = jnp.zeros((128, 16), jnp.float32).at[
        jnp.arange(128), jnp.arange(128) // 16].set(a1.reshape(-1))
    c = jnp.einsum('hfk,hk->fh', feat_W, a2)
    cpad = jnp.zeros((FEAT_PAD, 16), jnp.float32).at[:FEAT, :H].set(c)
    dvec = jnp.zeros((1, 16), jnp.float32).at[0, :H].set(
        jnp.einsum('hk,hk->h', feat_b, a2))
    ea_pad = jnp.pad(edge_attr, ((0, 0), (0, FEAT_PAD - FEAT)))
    expand = jnp.zeros((16, 128), jnp.float32).at[
        jnp.arange(128) // 16, jnp.arange(128)].set(1.0)

    zc, ns = _node_prep(e, wcat, a1m)
    ps = _edge_prep(ea_pad, cpad, dvec)

    zagg = jnp.zeros((N_SP, 128), jnp.float32)
    nsp128 = jnp.pad(ns, ((0, N_SP - N_E), (0, 0))).reshape(N_SP // 8, 128)
    ps1 = ps.reshape(-1)
    aggp, denp128 = _sc_edge(zc, nsp128, ps1, src, dst, zagg)
    denp = denp128.reshape(NC, N_SP, 16)

    return _tail(aggp[0, :N_S], aggp[1, :N_S], denp[0, :N_S], denp[1, :N_S],
                 s, expand,
                 w1, b1.reshape(1, FFN), w2, b2.reshape(1, 128),
                 ln_g.reshape(1, 128), ln_b.reshape(1, 128))


# speedup vs baseline: 7.7863x; 1.1038x over previous
import functools
import jax
import jax.numpy as jnp
import numpy as np
from jax import lax
from jax.experimental import pallas as pl
from jax.experimental.pallas import tpu as pltpu
from jax.experimental.pallas import tpu_sc as plsc

N_E = 10000
N_S = 10000
E = 320000
H = 8
DH = 16
FEAT = 50
FEAT_PAD = 64
FFN = 512

NC = 2
NSUB = 16
NW = NC * NSUB
EPW = E // NW
EB = 32
N_SP = 10240
ROWS_PER_TILE = N_SP // NSUB

NODE_BLK = 1000
EDGE_BLK = 3200


def _node_prep_body(e_ref, wcat_ref, a1m_ref, zc_ref, ns_ref):
    z = jnp.dot(e_ref[...], wcat_ref[...], preferred_element_type=jnp.float32)
    zc_ref[...] = z
    ns_ref[...] = jnp.dot(z, a1m_ref[...], preferred_element_type=jnp.float32)


def _node_prep(e, wcat, a1m):
    return pl.pallas_call(
        _node_prep_body,
        grid=(N_E // NODE_BLK,),
        in_specs=[
            pl.BlockSpec((NODE_BLK, 128), lambda i: (i, 0)),
            pl.BlockSpec((128, 128), lambda i: (0, 0)),
            pl.BlockSpec((128, 16), lambda i: (0, 0)),
        ],
        out_specs=[
            pl.BlockSpec((NODE_BLK, 128), lambda i: (i, 0)),
            pl.BlockSpec((NODE_BLK, 16), lambda i: (i, 0)),
        ],
        out_shape=[
            jax.ShapeDtypeStruct((N_E, 128), jnp.float32),
            jax.ShapeDtypeStruct((N_E, 16), jnp.float32),
        ],
    )(e, wcat, a1m)


def _edge_prep_body(ea_ref, c_ref, d_ref, ps_ref):
    ps_ref[...] = (
        jnp.dot(ea_ref[...], c_ref[...], preferred_element_type=jnp.float32)
        + d_ref[...]
    )


def _edge_prep(ea_pad, cpad, dvec):
    return pl.pallas_call(
        _edge_prep_body,
        grid=(E // EDGE_BLK,),
        in_specs=[
            pl.BlockSpec((EDGE_BLK, FEAT_PAD), lambda i: (i, 0)),
            pl.BlockSpec((FEAT_PAD, 16), lambda i: (0, 0)),
            pl.BlockSpec((1, 16), lambda i: (0, 0)),
        ],
        out_specs=pl.BlockSpec((EDGE_BLK, 16), lambda i: (i, 0)),
        out_shape=jax.ShapeDtypeStruct((E, 16), jnp.float32),
    )(ea_pad, cpad, dvec)


NBATCH = E // EB
NB_BASE = NBATCH // NW
NB_EXTRA = NBATCH - NB_BASE * NW


def _sc_edge_body(zc_hbm, nsp_hbm, ps1_hbm, src_hbm, dst_hbm, zagg_hbm,
                  aggp_hbm, denp_hbm,
                  srcb, srcb8, dstb, dstb8, zrows, nsg, psrows, wbuf,
                  agg_sh, den_sh, nst_sh, sem1, sem2):
    cid = lax.axis_index("c")
    sid = lax.axis_index("s")
    wid = cid * NSUB + sid
    zbase = sid * ROWS_PER_TILE
    pbase = sid * (ROWS_PER_TILE // 8)

    pltpu.sync_copy(zagg_hbm.at[pl.ds(zbase, ROWS_PER_TILE)],
                    agg_sh.at[pl.ds(zbase, ROWS_PER_TILE)])
    pltpu.sync_copy(zagg_hbm.at[pl.ds(zbase, ROWS_PER_TILE // 8)],
                    den_sh.at[pl.ds(pbase, ROWS_PER_TILE // 8)])
    pltpu.sync_copy(nsp_hbm.at[pl.ds(pbase, ROWS_PER_TILE // 8)],
                    nst_sh.at[pl.ds(pbase, ROWS_PER_TILE // 8)])

    def zw(i, c):
        for slot in range(8):
            wbuf[i, pl.ds(slot * 16, 16)] = jnp.zeros((16,), jnp.float32)
        return c
    lax.fori_loop(0, EB, zw, 0)
    plsc.subcore_barrier()

    nb = jnp.where(wid < NB_EXTRA, NB_BASE + 1, NB_BASE)

    def batch_body(t, carry):
        off = (t * NW + wid) * EB
        ca1 = pltpu.async_copy(src_hbm.at[pl.ds(off, EB)], srcb, sem1)
        ca2 = pltpu.async_copy(dst_hbm.at[pl.ds(off, EB)], dstb, sem2)
        ca3 = pltpu.async_copy(ps1_hbm.at[pl.ds(off * 16, EB * 16)], psrows, sem1)
        ca1.wait()
        ca2.wait()
        ca3.wait()
        for k in range(EB // 16):
            srcb8[pl.ds(k * 16, 16)] = lax.shift_right_logical(
                srcb[pl.ds(k * 16, 16)], 3)
            dstb8[pl.ds(k * 16, 16)] = lax.shift_right_logical(
                dstb[pl.ds(k * 16, 16)], 3)
        cp1 = pltpu.async_copy(zc_hbm.at[srcb], zrows, sem1)
        cp2 = pltpu.async_copy(nst_sh.at[srcb8], nsg, sem2)
        cp2.wait()

        def score_body(g, c1):
            sv = srcb[pl.ds(g * 16, 16)]
            dv = dstb[pl.ds(g * 16, 16)]
            for l in range(16):
                i = g * 16 + l
                soff = (sv[l] & 7) * 16
                doff = (dv[l] & 7) * 16
                nsv = nsg[i, pl.ds(soff, 16)]
                scv = nsv + psrows[pl.ds(i * 16, 16)]
                scv = jnp.where(scv >= 0.0, scv, 0.01 * scv)
                wbuf[i, pl.ds(doff, 16)] = jnp.exp(scv)
            return c1

        lax.fori_loop(0, EB // 16, score_body, 0)
        cp1.wait()

        def scale_body(g, c1):
            dv = dstb[pl.ds(g * 16, 16)]
            for l in range(16):
                i = g * 16 + l
                wv = wbuf[i, pl.ds((dv[l] & 7) * 16, 16)]
                for h in range(H):
                    seg = zrows[i, pl.ds(h * 16, 16)]
                    zrows[i, pl.ds(h * 16, 16)] = seg * wv[h]
            return c1

        lax.fori_loop(0, EB // 16, scale_body, 0)
        cs1 = pltpu.async_copy(zrows, agg_sh.at[dstb], sem1, add=True)
        cs2 = pltpu.async_copy(wbuf, den_sh.at[dstb8], sem2, add=True)
        cs1.wait()
        cs2.wait()

        def rz(g, c2):
            dv = dstb[pl.ds(g * 16, 16)]
            for l in range(16):
                wbuf[g * 16 + l, pl.ds((dv[l] & 7) * 16, 16)] = (
                    jnp.zeros((16,), jnp.float32))
            return c2

        lax.fori_loop(0, EB // 16, rz, 0)
        return carry

    lax.fori_loop(0, nb, batch_body, 0)
    plsc.subcore_barrier()

    pltpu.sync_copy(agg_sh.at[pl.ds(zbase, ROWS_PER_TILE)],
                    aggp_hbm.at[cid, pl.ds(zbase, ROWS_PER_TILE)])
    pltpu.sync_copy(den_sh.at[pl.ds(pbase, ROWS_PER_TILE // 8)],
                    denp_hbm.at[cid, pl.ds(pbase, ROWS_PER_TILE // 8)])


def _sc_edge(zc, nsp128, ps1, src, dst, zagg):
    mesh = plsc.VectorSubcoreMesh(core_axis_name="c", subcore_axis_name="s")
    fn = functools.partial(
        pl.kernel,
        mesh=mesh,
        out_type=[
            jax.ShapeDtypeStruct((NC, N_SP, 128), jnp.float32),
            jax.ShapeDtypeStruct((NC, N_SP // 8, 128), jnp.float32),
        ],
        scratch_types=[
            pltpu.VMEM((EB,), jnp.int32),
            pltpu.VMEM((EB,), jnp.int32),
            pltpu.VMEM((EB,), jnp.int32),
            pltpu.VMEM((EB,), jnp.int32),
            pltpu.VMEM((EB, 128), jnp.float32),
            pltpu.VMEM((EB, 128), jnp.float32),
            pltpu.VMEM((EB * 16,), jnp.float32),
            pltpu.VMEM((EB, 128), jnp.float32),
            pltpu.VMEM_SHARED((N_SP, 128), jnp.float32),
            pltpu.VMEM_SHARED((N_SP // 8, 128), jnp.float32),
            pltpu.VMEM_SHARED((N_SP // 8, 128), jnp.float32),
            pltpu.SemaphoreType.DMA,
            pltpu.SemaphoreType.DMA,
        ],
    )(_sc_edge_body)
    return fn(zc, nsp128, ps1, src, dst, zagg)


def _tail_body(a0_ref, a1_ref, d0_ref, d1_ref, s_ref, exp_ref,
               w1_ref, b1_ref, w2_ref, b2_ref, g_ref, b_ref, out_ref):
    aggu = a0_ref[...] + a1_ref[...]
    den = d0_ref[...] + d1_ref[...] + 1e-10
    den_exp = jnp.dot(den, exp_ref[...], preferred_element_type=jnp.float32)
    agg = aggu / den_exp
    hh = jnp.where(agg > 0.0, agg, jnp.exp(agg) - 1.0) + s_ref[...]
    mu = jnp.mean(hh, axis=-1, keepdims=True)
    var = jnp.mean((hh - mu) ** 2, axis=-1, keepdims=True)
    xn = (hh - mu) * lax.rsqrt(var + 1e-6) * g_ref[...] + b_ref[...]
    t = jnp.dot(xn, w1_ref[...], preferred_element_type=jnp.float32) + b1_ref[...]
    inter = t * 0.5 * (1.0 + lax.erf(t * np.float32(1.0 / np.sqrt(2.0))))
    out_ref[...] = (
        jnp.dot(inter, w2_ref[...], preferred_element_type=jnp.float32)
        + b2_ref[...] + hh
    )


def _tail(a0, a1, d0, d1, s, expand, w1, b1, w2, b2, ln_g, ln_b):
    return pl.pallas_call(
        _tail_body,
        grid=(N_S // NODE_BLK,),
        in_specs=[
            pl.BlockSpec((NODE_BLK, 128), lambda i: (i, 0)),
            pl.BlockSpec((NODE_BLK, 128), lambda i: (i, 0)),
            pl.BlockSpec((NODE_BLK, 16), lambda i: (i, 0)),
            pl.BlockSpec((NODE_BLK, 16), lambda i: (i, 0)),
            pl.BlockSpec((NODE_BLK, 128), lambda i: (i, 0)),
            pl.BlockSpec((16, 128), lambda i: (0, 0)),
            pl.BlockSpec((128, FFN), lambda i: (0, 0)),
            pl.BlockSpec((1, FFN), lambda i: (0, 0)),
            pl.BlockSpec((FFN, 128), lambda i: (0, 0)),
            pl.BlockSpec((1, 128), lambda i: (0, 0)),
            pl.BlockSpec((1, 128), lambda i: (0, 0)),
            pl.BlockSpec((1, 128), lambda i: (0, 0)),
        ],
        out_specs=pl.BlockSpec((NODE_BLK, 128), lambda i: (i, 0)),
        out_shape=jax.ShapeDtypeStruct((N_S, 128), jnp.float32),
    )(a0, a1, d0, d1, s, expand, w1, b1, w2, b2, ln_g, ln_b)


def kernel(e, s, edge_index, edge_attr, fc_W, attn_W, feat_W, feat_b,
           w1, b1, w2, b2, ln_g, ln_b):
    src = edge_index[0].astype(jnp.int32)
    dst = edge_index[1].astype(jnp.int32)
    a1 = attn_W[:, :DH]
    a2 = attn_W[:, DH:]

    wcat = jnp.transpose(fc_W, (1, 0, 2)).reshape(128, H * DH)
    a1m = jnp.zeros((128, 16), jnp.float32).at[
        jnp.arange(128), jnp.arange(128) // 16].set(a1.reshape(-1))
    c = jnp.einsum('hfk,hk->fh', feat_W, a2)
    cpad = jnp.zeros((FEAT_PAD, 16), jnp.float32).at[:FEAT, :H].set(c)
    dvec = jnp.zeros((1, 16), jnp.float32).at[0, :H].set(
        jnp.einsum('hk,hk->h', feat_b, a2))
    ea_pad = jnp.pad(edge_attr, ((0, 0), (0, FEAT_PAD - FEAT)))
    expand = jnp.zeros((16, 128), jnp.float32).at[
        jnp.arange(128) // 16, jnp.arange(128)].set(1.0)

    zc, ns = _node_prep(e, wcat, a1m)
    ps = _edge_prep(ea_pad, cpad, dvec)

    zagg = jnp.zeros((N_SP, 128), jnp.float32)
    nsp128 = jnp.pad(ns, ((0, N_SP - N_E), (0, 0))).reshape(N_SP // 8, 128)
    ps1 = ps.reshape(-1)
    aggp, denp128 = _sc_edge(zc, nsp128, ps1, src, dst, zagg)
    denp = denp128.reshape(NC, N_SP, 16)

    return _tail(aggp[0, :N_S], aggp[1, :N_S], denp[0, :N_S], denp[1, :N_S],
                 s, expand,
                 w1, b1.reshape(1, FFN), w2, b2.reshape(1, 128),
                 ln_g.reshape(1, 128), ln_b.reshape(1, 128))
